# baseline (speedup 1.0000x reference)
import functools

import jax
import jax.numpy as jnp
from jax import lax
from jax.experimental import pallas as pl
from jax.experimental.pallas import tpu as pltpu, tpu_sc as plsc

VOCAB = 49408
MAX_POS = 77
D_MODEL = 768
BATCH = 1024
SEQ = 77

_LANES = 16
_CHUNK = 32
_N_FLAT = BATCH * SEQ
_PIECES = D_MODEL // 128
_PHYS_ROWS = _N_FLAT * _PIECES
_CHUNK_PHYS = _CHUNK * _PIECES


def _make_kernel():
    info = plsc.get_sparse_core_info()
    nc, ns = info.num_cores, info.num_subcores
    nw = nc * ns
    rows_per_w = _N_FLAT // nw
    n_chunks = rows_per_w // _CHUNK

    mesh = plsc.VectorSubcoreMesh(core_axis_name="c", subcore_axis_name="s")

    @functools.partial(
        pl.kernel,
        mesh=mesh,
        out_type=jax.ShapeDtypeStruct((_PHYS_ROWS, 128), jnp.float32),
        scratch_types=[
            pltpu.VMEM((n_chunks, _CHUNK), jnp.int32),
            pltpu.VMEM((4, D_MODEL), jnp.float32),
            pltpu.VMEM((_CHUNK, D_MODEL), jnp.float32),
            pltpu.VMEM((_CHUNK, D_MODEL), jnp.float32),
            pltpu.VMEM((_CHUNK_PHYS, 128), jnp.float32),
            pltpu.VMEM((_CHUNK_PHYS, 128), jnp.float32),
            pltpu.SemaphoreType.DMA,
            pltpu.SemaphoreType.DMA,
            pltpu.SemaphoreType.DMA,
            pltpu.SemaphoreType.DMA,
        ],
        compiler_params=pltpu.CompilerParams(use_tc_tiling_on_sc=False),
    )
    def emb(ids_hbm, tok_hbm, pos_hbm, out_hbm,
            idx_v, pos_v, g0, g1, s0, s1, sg0, sg1, ss0, ss1):
        wid = lax.axis_index("s") * nc + lax.axis_index("c")
        base = wid * rows_per_w
        p_lo = lax.min(base >> 10, jnp.int32(MAX_POS - 4))

        pltpu.sync_copy(ids_hbm.at[wid], idx_v)
        pltpu.sync_copy(pos_hbm.at[pl.ds(p_lo, 4)], pos_v)

        gbufs = (g0, g1)
        sbufs = (s0, s1)
        sgs = (sg0, sg1)
        sss = (ss0, ss1)

        def start_gather(k, b):
            pltpu.async_copy(tok_hbm.at[idx_v.at[k]], gbufs[b], sgs[b])

        def wait_gather(k, b):
            pltpu.make_async_copy(tok_hbm.at[idx_v.at[k]], gbufs[b],
                                  sgs[b]).wait()

        def out_slice(k):
            q = base + k * _CHUNK
            p = q >> 10
            i0 = (q & 1023) >> 3
            r0 = p * (128 * 48) + i0 * 48
            return out_hbm.at[pl.ds(r0, _CHUNK_PHYS)]

        def start_scatter(k, b):
            pltpu.async_copy(sbufs[b], out_slice(k), sss[b])

        def wait_scatter(k, b):
            pltpu.make_async_copy(sbufs[b], out_slice(k), sss[b]).wait()

        def add_chunk(k, b):
            gbuf, sbuf = gbufs[b], sbufs[b]
            q = base + k * _CHUNK
            p_idx = (q >> 10) - p_lo
            for jg in range(2):
                posv = [pos_v[p_idx, pl.ds(jg * 384 + 16 * t, _LANES)]
                        for t in range(24)]

                def row_body(ri, carry):
                    srow0 = (ri >> 3) * 48 + (ri & 7) + jg * 24
                    for t in range(24):
                        j_sub, c_sub = divmod(t, 8)
                        sbuf[srow0 + 8 * j_sub, pl.ds(16 * c_sub, _LANES)] = (
                            gbuf[ri, pl.ds(jg * 384 + 16 * t, _LANES)]
                            + posv[t])
                    return carry
                lax.fori_loop(0, _CHUNK, row_body, 0)

        start_gather(0, 0)
        start_gather(1, 1)
        wait_gather(0, 0)
        add_chunk(0, 0)
        start_scatter(0, 0)
        start_gather(2, 0)
        wait_gather(1, 1)
        add_chunk(1, 1)
        start_scatter(1, 1)
        start_gather(3, 1)

        def pair(k2, carry):
            for b in range(2):
                k = 2 * k2 + b
                wait_gather(k, b)
                wait_scatter(k - 2, b)
                add_chunk(k, b)
                start_scatter(k, b)
                start_gather(k + 2, b)
            return carry
        lax.fori_loop(1, 37, pair, jnp.int32(0))

        wait_gather(74, 0)
        wait_scatter(72, 0)
        add_chunk(74, 0)
        start_scatter(74, 0)
        start_gather(76, 0)

        wait_gather(75, 1)
        wait_scatter(73, 1)
        add_chunk(75, 1)
        start_scatter(75, 1)

        wait_gather(76, 0)
        wait_scatter(74, 0)
        add_chunk(76, 0)
        start_scatter(76, 0)

        wait_scatter(75, 1)
        wait_scatter(76, 0)

    return emb


_emb_kernel = _make_kernel()


def kernel(input_ids, token_embedding, position_embedding):
    nw = 32
    ids = input_ids.astype(jnp.int32).T.reshape(
        nw, _N_FLAT // nw // _CHUNK, _CHUNK)
    out = _emb_kernel(ids, token_embedding, position_embedding)
    arr5 = out.reshape(SEQ, BATCH // 8, _PIECES, 8, 128)
    return arr5.transpose(1, 3, 0, 2, 4).reshape(BATCH, SEQ, D_MODEL)


# speedup vs baseline: 2.1113x; 2.1113x over previous
import functools

import jax
import jax.numpy as jnp
from jax import lax
from jax.experimental import pallas as pl
from jax.experimental.pallas import tpu as pltpu, tpu_sc as plsc

VOCAB = 49408
MAX_POS = 77
D_MODEL = 768
BATCH = 1024
SEQ = 77

_LANES = 16
_CHUNK = 32
_N_FLAT = BATCH * SEQ
_PIECES = D_MODEL // 128
_PHYS_ROWS = _N_FLAT * _PIECES
_CHUNK_PHYS = _CHUNK * _PIECES


def _make_kernel():
    info = plsc.get_sparse_core_info()
    nc, ns = info.num_cores, info.num_subcores
    nw = nc * ns
    rows_per_w = _N_FLAT // nw
    n_chunks = rows_per_w // _CHUNK

    mesh = plsc.VectorSubcoreMesh(core_axis_name="c", subcore_axis_name="s")

    @functools.partial(
        pl.kernel,
        mesh=mesh,
        out_type=jax.ShapeDtypeStruct((_PHYS_ROWS, 128), jnp.float32),
        scratch_types=[
            pltpu.VMEM((n_chunks, _CHUNK), jnp.int32),
            pltpu.VMEM((4, D_MODEL), jnp.float32),
            pltpu.VMEM((_CHUNK, D_MODEL), jnp.float32),
            pltpu.VMEM((_CHUNK, D_MODEL), jnp.float32),
            pltpu.VMEM((_CHUNK, D_MODEL), jnp.float32),
            pltpu.VMEM((_CHUNK_PHYS, 128), jnp.float32),
            pltpu.VMEM((_CHUNK_PHYS, 128), jnp.float32),
            pltpu.SemaphoreType.DMA,
            pltpu.SemaphoreType.DMA,
            pltpu.SemaphoreType.DMA,
            pltpu.SemaphoreType.DMA,
            pltpu.SemaphoreType.DMA,
        ],
        compiler_params=pltpu.CompilerParams(use_tc_tiling_on_sc=False),
    )
    def emb(ids_hbm, tok_hbm, pos_hbm, out_hbm,
            idx_v, pos_v, g0, g1, g2, s0, s1, sg0, sg1, sg2, ss0, ss1):
        wid = lax.axis_index("s") * nc + lax.axis_index("c")
        base = wid * rows_per_w
        p_lo = lax.min(base >> 10, jnp.int32(MAX_POS - 4))

        pltpu.sync_copy(ids_hbm.at[wid], idx_v)
        pltpu.sync_copy(pos_hbm.at[pl.ds(p_lo, 4)], pos_v)

        gbufs = (g0, g1, g2)
        sbufs = (s0, s1)
        sgs = (sg0, sg1, sg2)
        sss = (ss0, ss1)

        def start_gather(k, b):
            pltpu.async_copy(tok_hbm.at[idx_v.at[k]], gbufs[b], sgs[b])

        def wait_gather(k, b):
            pltpu.make_async_copy(tok_hbm.at[idx_v.at[k]], gbufs[b],
                                  sgs[b]).wait()

        def out_slice(k):
            q = base + k * _CHUNK
            p = q >> 10
            i0 = (q & 1023) >> 3
            r0 = p * (128 * 48) + i0 * 48
            return out_hbm.at[pl.ds(r0, _CHUNK_PHYS)]

        def start_scatter(k, b):
            pltpu.async_copy(sbufs[b], out_slice(k), sss[b])

        def wait_scatter(k, b):
            pltpu.make_async_copy(sbufs[b], out_slice(k), sss[b]).wait()

        def add_chunk(k, bg, bs):
            gbuf, sbuf = gbufs[bg], sbufs[bs]
            q = base + k * _CHUNK
            p_idx = (q >> 10) - p_lo
            def jg_body(jg, carry):
                posv = [pos_v[p_idx, pl.ds(jg * 384 + 16 * t, _LANES)]
                        for t in range(24)]

                def row_body(ri):
                    srow0 = (ri >> 3) * 48 + (ri & 7) + jg * 24
                    for t in range(24):
                        j_sub, c_sub = divmod(t, 8)
                        sbuf[srow0 + 8 * j_sub, pl.ds(16 * c_sub, _LANES)] = (
                            gbuf[ri, pl.ds(jg * 384 + 16 * t, _LANES)]
                            + posv[t])
                plsc.parallel_loop(0, _CHUNK, unroll=2)(row_body)
                return carry
            lax.fori_loop(0, 2, jg_body, 0)

        start_gather(0, 0)
        start_gather(1, 1)
        start_gather(2, 2)

        wait_gather(0, 0)
        add_chunk(0, 0, 0)
        start_scatter(0, 0)
        start_gather(3, 0)

        wait_gather(1, 1)
        add_chunk(1, 1, 1)
        start_scatter(1, 1)
        start_gather(4, 1)

        def superstep(it, carry):
            for b in range(6):
                k = 2 + 6 * it + b
                bg = (2 + b) % 3
                bs = b % 2
                wait_gather(k, bg)
                wait_scatter(k - 2, bs)
                add_chunk(k, bg, bs)
                start_scatter(k, bs)
                start_gather(k + 3, bg)
            return carry
        lax.fori_loop(0, 12, superstep, jnp.int32(0))

        wait_gather(74, 2)
        wait_scatter(72, 0)
        add_chunk(74, 2, 0)
        start_scatter(74, 0)

        wait_gather(75, 0)
        wait_scatter(73, 1)
        add_chunk(75, 0, 1)
        start_scatter(75, 1)

        wait_gather(76, 1)
        wait_scatter(74, 0)
        add_chunk(76, 1, 0)
        start_scatter(76, 0)

        wait_scatter(75, 1)
        wait_scatter(76, 0)

    return emb


_emb_kernel = _make_kernel()


def kernel(input_ids, token_embedding, position_embedding):
    nw = 32
    ids = input_ids.astype(jnp.int32).T.reshape(
        nw, _N_FLAT // nw // _CHUNK, _CHUNK)
    out = _emb_kernel(ids, token_embedding, position_embedding)
    arr5 = out.reshape(SEQ, BATCH // 8, _PIECES, 8, 128)
    return arr5.transpose(1, 3, 0, 2, 4).reshape(BATCH, SEQ, D_MODEL)
